# baseline (speedup 1.0000x reference)
import functools

import jax
import jax.numpy as jnp
from jax import lax
from jax.experimental import pallas as pl
from jax.experimental.pallas import tpu as pltpu
from jax.experimental.pallas import tpu_sc as plsc

_EXPERTS = 64
_BLOCK_T = 4096
_N_WORKERS = 32


def _gate_block(x_ref, w_ref, logits_ref, stats_ref):
    x = x_ref[...]
    w = w_ref[...]
    logits = jax.lax.dot_general(
        x, w, (((1,), (1,)), ((), ())), preferred_element_type=jnp.float32
    )
    logits_ref[...] = logits

    m = jnp.max(logits, axis=1, keepdims=True)
    e = jnp.exp(logits - m)
    s = jnp.sum(e, axis=1, keepdims=True)

    col = jax.lax.broadcasted_iota(jnp.int32, logits.shape, 1).astype(jnp.float32)
    i1 = jnp.min(jnp.where(logits == m, col, jnp.float32(_EXPERTS)),
                 axis=1, keepdims=True)
    l2 = jnp.where(col == i1, jnp.float32(-jnp.inf), logits)
    m2 = jnp.max(l2, axis=1, keepdims=True)
    i2 = jnp.min(jnp.where(l2 == m2, col, jnp.float32(_EXPERTS)),
                 axis=1, keepdims=True)
    p1 = 1.0 / s
    p2 = jnp.exp(m2 - m) / s
    stats_ref[...] = jnp.concatenate([i1, i2, p1, p2], axis=1)


def _tc_logits_stats(x, W):
    n_tokens = x.shape[0]
    grid = (n_tokens // _BLOCK_T,)
    return pl.pallas_call(
        _gate_block,
        grid=grid,
        in_specs=[
            pl.BlockSpec((_BLOCK_T, x.shape[1]), lambda i: (i, 0)),
            pl.BlockSpec((W.shape[0], W.shape[1]), lambda i: (0, 0)),
        ],
        out_specs=[
            pl.BlockSpec((_BLOCK_T, _EXPERTS), lambda i: (i, 0)),
            pl.BlockSpec((_BLOCK_T, 4), lambda i: (i, 0)),
        ],
        out_shape=[
            jax.ShapeDtypeStruct((n_tokens, _EXPERTS), jnp.float32),
            jax.ShapeDtypeStruct((n_tokens, 4), jnp.float32),
        ],
    )(x, W)


def _sc_scatter_y(stats):
    n_tokens = stats.shape[0]
    rows_per_w = n_tokens // _N_WORKERS

    def body(stats_hbm, y_hbm, stats_v, y_v):
        c = lax.axis_index("c")
        s = lax.axis_index("s")
        wid = s * 2 + c
        base = wid * rows_per_w
        pltpu.sync_copy(stats_hbm.at[pl.ds(base, rows_per_w)], stats_v)

        zeros16 = jnp.zeros((16,), jnp.float32)

        def zero_body(i, carry):
            y_v[i, pl.ds(0, 16)] = zeros16
            y_v[i, pl.ds(16, 16)] = zeros16
            y_v[i, pl.ds(32, 16)] = zeros16
            y_v[i, pl.ds(48, 16)] = zeros16
            return carry

        lax.fori_loop(0, rows_per_w, zero_body, 0)

        iota = lax.broadcasted_iota(jnp.int32, (16,), 0)
        zeroi = jnp.zeros((16,), jnp.int32)

        def grp_body(g, carry):
            rows = g * 16 + iota
            i1 = plsc.load_gather(stats_v, [rows, zeroi])
            i2 = plsc.load_gather(stats_v, [rows, zeroi + 1])
            p1 = plsc.load_gather(stats_v, [rows, zeroi + 2])
            p2 = plsc.load_gather(stats_v, [rows, zeroi + 3])
            plsc.store_scatter(y_v, [rows, i1.astype(jnp.int32)], p1)
            plsc.store_scatter(y_v, [rows, i2.astype(jnp.int32)], p2)
            return carry

        lax.fori_loop(0, rows_per_w // 16, grp_body, 0)

        pltpu.sync_copy(y_v, y_hbm.at[pl.ds(base, rows_per_w)])

    return pl.kernel(
        body,
        out_type=jax.ShapeDtypeStruct((n_tokens, _EXPERTS), jnp.float32),
        mesh=plsc.VectorSubcoreMesh(core_axis_name="c", subcore_axis_name="s"),
        compiler_params=pltpu.CompilerParams(use_tc_tiling_on_sc=False, needs_layout_passes=False),
        scratch_types=[
            pltpu.VMEM((rows_per_w, 4), jnp.float32),
            pltpu.VMEM((rows_per_w, _EXPERTS), jnp.float32),
        ],
    )(stats)


def kernel(x, W):
    logits, stats = _tc_logits_stats(x, W)
    y = _sc_scatter_y(stats)
    return (y, logits)


# speedup vs baseline: 1.1087x; 1.1087x over previous
import jax
import jax.numpy as jnp
from jax.experimental import pallas as pl
from jax.experimental.pallas import tpu as pltpu

_EXPERTS = 64
_BLOCK_T = 4096
_KSPLIT = 6


def _gate_block(x_ref, w_ref, y_ref, logits_ref, acc_ref):
    k = pl.program_id(1)
    part = jax.lax.dot_general(
        x_ref[...], w_ref[...], (((1,), (1,)), ((), ())),
        preferred_element_type=jnp.float32,
    )

    @pl.when(k == 0)
    def _init():
        acc_ref[...] = part

    @pl.when(k > 0)
    def _accum():
        acc_ref[...] += part

    @pl.when(k == _KSPLIT - 1)
    def _epilogue():
        logits = acc_ref[...]
        logits_ref[...] = logits
        m = jnp.max(logits, axis=1, keepdims=True)
        e = jnp.exp(logits - m)
        s = jnp.sum(e, axis=1, keepdims=True)
        col = jax.lax.broadcasted_iota(jnp.int32, logits.shape, 1).astype(
            jnp.float32)
        i1 = jnp.min(jnp.where(logits == m, col, jnp.float32(_EXPERTS)),
                     axis=1, keepdims=True)
        at1 = col == i1
        l2 = jnp.where(at1, jnp.float32(-jnp.inf), logits)
        m2 = jnp.max(l2, axis=1, keepdims=True)
        keep = at1 | (l2 == m2)
        y_ref[...] = jnp.where(keep, e / s, jnp.float32(0.0))


def kernel(x, W):
    n_tokens, k_dim = x.shape
    bk = k_dim // _KSPLIT
    grid = (n_tokens // _BLOCK_T, _KSPLIT)
    y, logits = pl.pallas_call(
        _gate_block,
        grid=grid,
        in_specs=[
            pl.BlockSpec((_BLOCK_T, bk), lambda i, k: (i, k)),
            pl.BlockSpec((W.shape[0], bk), lambda i, k: (0, k)),
        ],
        out_specs=[
            pl.BlockSpec((_BLOCK_T, _EXPERTS), lambda i, k: (i, 0)),
            pl.BlockSpec((_BLOCK_T, _EXPERTS), lambda i, k: (i, 0)),
        ],
        out_shape=[
            jax.ShapeDtypeStruct((n_tokens, _EXPERTS), jnp.float32),
            jax.ShapeDtypeStruct((n_tokens, _EXPERTS), jnp.float32),
        ],
        scratch_shapes=[pltpu.VMEM((_BLOCK_T, _EXPERTS), jnp.float32)],
    )(x, W)
    return (y, logits)


# speedup vs baseline: 1.7428x; 1.5719x over previous
import jax
import jax.numpy as jnp
from jax.experimental import pallas as pl

_EXPERTS = 64
_BLOCK_T = 2048


def _gate_block(x_ref, w_ref, y_ref, logits_ref):
    logits = jax.lax.dot_general(
        x_ref[...], w_ref[...], (((1,), (1,)), ((), ())),
        preferred_element_type=jnp.float32,
    )
    logits_ref[...] = logits
    m = jnp.max(logits, axis=1, keepdims=True)
    e = jnp.exp(logits - m)
    s = jnp.sum(e, axis=1, keepdims=True)
    col = jax.lax.broadcasted_iota(jnp.int32, logits.shape, 1).astype(
        jnp.float32)
    i1 = jnp.min(jnp.where(logits == m, col, jnp.float32(_EXPERTS)),
                 axis=1, keepdims=True)
    at1 = col == i1
    l2 = jnp.where(at1, jnp.float32(-jnp.inf), logits)
    m2 = jnp.max(l2, axis=1, keepdims=True)
    keep = at1 | (l2 == m2)
    y_ref[...] = jnp.where(keep, e / s, jnp.float32(0.0))


def kernel(x, W):
    n_tokens, k_dim = x.shape
    grid = (n_tokens // _BLOCK_T,)
    y, logits = pl.pallas_call(
        _gate_block,
        grid=grid,
        in_specs=[
            pl.BlockSpec((_BLOCK_T, k_dim), lambda i: (i, 0)),
            pl.BlockSpec(W.shape, lambda i: (0, 0)),
        ],
        out_specs=[
            pl.BlockSpec((_BLOCK_T, _EXPERTS), lambda i: (i, 0)),
            pl.BlockSpec((_BLOCK_T, _EXPERTS), lambda i: (i, 0)),
        ],
        out_shape=[
            jax.ShapeDtypeStruct((n_tokens, _EXPERTS), jnp.float32),
            jax.ShapeDtypeStruct((n_tokens, _EXPERTS), jnp.float32),
        ],
    )(x, W)
    return (y, logits)


# speedup vs baseline: 1.8059x; 1.0362x over previous
import jax
import jax.numpy as jnp
from jax.experimental import pallas as pl

_EXPERTS = 64
_BLOCK_T = 4096


def _gate_block(x_ref, w_ref, y_ref, logits_ref):
    logits = jax.lax.dot_general(
        x_ref[...], w_ref[...], (((1,), (1,)), ((), ())),
        preferred_element_type=jnp.float32,
    )
    logits_ref[...] = logits
    m = jnp.max(logits, axis=1, keepdims=True)
    e = jnp.exp(logits - m)
    s = jnp.sum(e, axis=1, keepdims=True)
    col = jax.lax.broadcasted_iota(jnp.int32, logits.shape, 1).astype(
        jnp.float32)
    i1 = jnp.min(jnp.where(logits == m, col, jnp.float32(_EXPERTS)),
                 axis=1, keepdims=True)
    at1 = col == i1
    l2 = jnp.where(at1, jnp.float32(-jnp.inf), logits)
    m2 = jnp.max(l2, axis=1, keepdims=True)
    keep = at1 | (l2 == m2)
    y_ref[...] = jnp.where(keep, e / s, jnp.float32(0.0))


def kernel(x, W):
    n_tokens, k_dim = x.shape
    grid = (n_tokens // _BLOCK_T,)
    y, logits = pl.pallas_call(
        _gate_block,
        grid=grid,
        in_specs=[
            pl.BlockSpec((_BLOCK_T, k_dim), lambda i: (i, 0)),
            pl.BlockSpec(W.shape, lambda i: (0, 0)),
        ],
        out_specs=[
            pl.BlockSpec((_BLOCK_T, _EXPERTS), lambda i: (i, 0)),
            pl.BlockSpec((_BLOCK_T, _EXPERTS), lambda i: (i, 0)),
        ],
        out_shape=[
            jax.ShapeDtypeStruct((n_tokens, _EXPERTS), jnp.float32),
            jax.ShapeDtypeStruct((n_tokens, _EXPERTS), jnp.float32),
        ],
    )(x, W)
    return (y, logits)
